# baseline (speedup 1.0000x reference)
import functools

import jax
import jax.numpy as jnp
from jax import lax
from jax.experimental import pallas as pl
from jax.experimental.pallas import tpu as pltpu
from jax.experimental.pallas import tpu_sc as plsc

BATCH = 16384
EMBED = 64
NC = 2
NS = 16
NW = NC * NS
B_PER_W = BATCH // NW
CHUNK = 128
N_CHUNKS = B_PER_W // CHUNK
IDX_ROWS = BATCH // CHUNK


def _sc_gather(u_ids2, i_ids2, user_table, item_table):
  mesh = plsc.VectorSubcoreMesh(core_axis_name="c", subcore_axis_name="s")

  @functools.partial(
      pl.kernel,
      out_type=(
          jax.ShapeDtypeStruct((BATCH, EMBED), jnp.float32),
          jax.ShapeDtypeStruct((BATCH, EMBED), jnp.float32),
      ),
      mesh=mesh,
      compiler_params=pltpu.CompilerParams(use_tc_tiling_on_sc=False),
      scratch_types=[
          pltpu.VMEM((N_CHUNKS, CHUNK), jnp.int32),
          pltpu.VMEM((N_CHUNKS, CHUNK), jnp.int32),
          pltpu.VMEM((B_PER_W, EMBED), jnp.float32),
          pltpu.VMEM((B_PER_W, EMBED), jnp.float32),
          pltpu.SemaphoreType.DMA,
      ],
  )
  def k(uids_hbm, iids_hbm, utab_hbm, itab_hbm, out_u, out_i,
        idx_u, idx_i, rows_u, rows_i, sem):
    wid = lax.axis_index("s") * NC + lax.axis_index("c")
    idx_base = wid * N_CHUNKS
    pltpu.sync_copy(uids_hbm.at[pl.ds(idx_base, N_CHUNKS)], idx_u)
    pltpu.sync_copy(iids_hbm.at[pl.ds(idx_base, N_CHUNKS)], idx_i)
    copies = []
    for j in range(N_CHUNKS):
      copies.append(pltpu.async_copy(
          utab_hbm.at[idx_u.at[j]], rows_u.at[pl.ds(j * CHUNK, CHUNK)], sem))
      copies.append(pltpu.async_copy(
          itab_hbm.at[idx_i.at[j]], rows_i.at[pl.ds(j * CHUNK, CHUNK)], sem))
    for c in copies:
      c.wait()
    base = wid * B_PER_W
    pltpu.sync_copy(rows_u, out_u.at[pl.ds(base, B_PER_W)])
    pltpu.sync_copy(rows_i, out_i.at[pl.ds(base, B_PER_W)])

  return k(u_ids2, i_ids2, user_table, item_table)


def _mlp_body(u, i, w1u, w1i, b1, w2, b2, w3, b3, out):
  f32 = jnp.float32
  hi = jax.lax.Precision.HIGHEST
  h = (jnp.dot(u[...], w1u[...], preferred_element_type=f32, precision=hi)
       + jnp.dot(i[...], w1i[...], preferred_element_type=f32, precision=hi)
       + b1[...])
  h = jnp.maximum(h, 0.0)
  h2 = jnp.dot(h, w2[...], preferred_element_type=f32, precision=hi) + b2[...]
  h2 = jnp.maximum(h2, 0.0)
  out[...] = jnp.dot(h2, w3[...], preferred_element_type=f32,
                     precision=hi) + b3[...]


def _mlp(u_rows, i_rows, W1u, W1i, b1, W2, b2, W3, b3, bm=2048):
  grid = (BATCH // bm,)
  full = lambda shape: pl.BlockSpec(shape, lambda m: (0,) * len(shape))
  return pl.pallas_call(
      _mlp_body,
      grid=grid,
      in_specs=[
          pl.BlockSpec((bm, EMBED), lambda m: (m, 0)),
          pl.BlockSpec((bm, EMBED), lambda m: (m, 0)),
          full((EMBED, 128)),
          full((EMBED, 128)),
          full((1, 128)),
          full((128, 64)),
          full((1, 64)),
          full((EMBED, 1)),
          full((1, 1)),
      ],
      out_specs=pl.BlockSpec((bm, 1), lambda m: (m, 0)),
      out_shape=jax.ShapeDtypeStruct((BATCH, 1), jnp.float32),
  )(u_rows, i_rows, W1u, W1i, b1, W2, b2, W3, b3)


def kernel(user_ids, item_ids, user_table, item_table, W1, b1, W2, b2, W3, b3):
  u_ids2 = user_ids.astype(jnp.int32).reshape(IDX_ROWS, CHUNK)
  i_ids2 = item_ids.astype(jnp.int32).reshape(IDX_ROWS, CHUNK)
  u_rows, i_rows = _sc_gather(u_ids2, i_ids2, user_table, item_table)
  return _mlp(u_rows, i_rows,
              W1[:EMBED], W1[EMBED:], b1.reshape(1, -1),
              W2, b2.reshape(1, -1), W3, b3.reshape(1, 1))


# speedup vs baseline: 1.5897x; 1.5897x over previous
import functools

import jax
import jax.numpy as jnp
from jax import lax
from jax.experimental import pallas as pl
from jax.experimental.pallas import tpu as pltpu
from jax.experimental.pallas import tpu_sc as plsc

BATCH = 16384
EMBED = 64
NC = 2
NS = 16
NW = NC * NS
B_PER_W = BATCH // NW
CHUNK = 128
N_CHUNKS = B_PER_W // CHUNK
PAIR = 2048


def _pack_body(ta_ref, tb_ref, out_ref):
  out_ref[...] = jnp.concatenate([ta_ref[...].T, tb_ref[...].T], axis=1)


def _pack(tab_t):
  n = tab_t.shape[1]
  nb = (n + 2 * PAIR - 1) // (2 * PAIR)
  last = (n + PAIR - 1) // PAIR - 1
  return pl.pallas_call(
      _pack_body,
      grid=(nb,),
      in_specs=[
          pl.BlockSpec((EMBED, PAIR), lambda m: (0, 2 * m)),
          pl.BlockSpec((EMBED, PAIR),
                       lambda m: (0, jnp.minimum(2 * m + 1, last))),
      ],
      out_specs=pl.BlockSpec((PAIR, 128), lambda m: (m, 0)),
      out_shape=jax.ShapeDtypeStruct((nb * PAIR, 128), jnp.float32),
  )(tab_t, tab_t)


def _sc_gather(qu2, qi2, upair, ipair):
  mesh = plsc.VectorSubcoreMesh(core_axis_name="c", subcore_axis_name="s")

  @functools.partial(
      pl.kernel,
      out_type=(
          jax.ShapeDtypeStruct((BATCH, 128), jnp.float32),
          jax.ShapeDtypeStruct((BATCH, 128), jnp.float32),
      ),
      mesh=mesh,
      scratch_types=[
          pltpu.VMEM((B_PER_W,), jnp.int32),
          pltpu.VMEM((B_PER_W, 128), jnp.float32),
          pltpu.SemaphoreType.DMA,
      ],
  )
  def k(qu_hbm, qi_hbm, up_hbm, ip_hbm, out_u, out_i, idx, rows, sem):
    wid = lax.axis_index("s") * NC + lax.axis_index("c")
    base = wid * B_PER_W
    for ids_hbm, pair_hbm, out in ((qu_hbm, up_hbm, out_u),
                                   (qi_hbm, ip_hbm, out_i)):
      pltpu.sync_copy(ids_hbm.at[pl.ds(base, B_PER_W)], idx)
      copies = []
      for j in range(B_PER_W // 16):
        iv = idx[pl.ds(j * 16, 16)]
        copies.append(pltpu.async_copy(
            pair_hbm.at[iv], rows.at[pl.ds(j * 16, 16)], sem))
      for c in copies:
        c.wait()
      pltpu.sync_copy(rows, out.at[pl.ds(base, B_PER_W)])

  return k(qu2, qi2, upair, ipair)


def _mlp_body(u, i, hu, hi, w1u, w1i, b1, w2, b2, w3, b3, out):
  f32 = jnp.float32
  hp = jax.lax.Precision.HIGHEST
  xu = jnp.where(hu[...] > 0.5, u[...][:, EMBED:], u[...][:, :EMBED])
  xi = jnp.where(hi[...] > 0.5, i[...][:, EMBED:], i[...][:, :EMBED])
  h = (jnp.dot(xu, w1u[...], preferred_element_type=f32, precision=hp)
       + jnp.dot(xi, w1i[...], preferred_element_type=f32, precision=hp)
       + b1[...])
  h = jnp.maximum(h, 0.0)
  h2 = jnp.dot(h, w2[...], preferred_element_type=f32, precision=hp) + b2[...]
  h2 = jnp.maximum(h2, 0.0)
  out[...] = jnp.dot(h2, w3[...], preferred_element_type=f32,
                     precision=hp) + b3[...]


def _mlp(u_pr, i_pr, hu, hi, W1u, W1i, b1, W2, b2, W3, b3, bm=2048):
  grid = (BATCH // bm,)
  full = lambda shape: pl.BlockSpec(shape, lambda m: (0,) * len(shape))
  return pl.pallas_call(
      _mlp_body,
      grid=grid,
      in_specs=[
          pl.BlockSpec((bm, 128), lambda m: (m, 0)),
          pl.BlockSpec((bm, 128), lambda m: (m, 0)),
          pl.BlockSpec((bm, 1), lambda m: (m, 0)),
          pl.BlockSpec((bm, 1), lambda m: (m, 0)),
          full((EMBED, 128)),
          full((EMBED, 128)),
          full((1, 128)),
          full((128, 64)),
          full((1, 64)),
          full((EMBED, 1)),
          full((1, 1)),
      ],
      out_specs=pl.BlockSpec((bm, 1), lambda m: (m, 0)),
      out_shape=jax.ShapeDtypeStruct((BATCH, 1), jnp.float32),
  )(u_pr, i_pr, hu, hi, W1u, W1i, b1, W2, b2, W3, b3)


def kernel(user_ids, item_ids, user_table, item_table, W1, b1, W2, b2, W3, b3):
  uid = user_ids.astype(jnp.int32)
  iid = item_ids.astype(jnp.int32)
  qu = (uid // (2 * PAIR)) * PAIR + uid % PAIR
  qi = (iid // (2 * PAIR)) * PAIR + iid % PAIR
  hu = ((uid // PAIR) % 2).astype(jnp.float32).reshape(-1, 1)
  hi = ((iid // PAIR) % 2).astype(jnp.float32).reshape(-1, 1)
  upair = _pack(user_table.T)
  ipair = _pack(item_table.T)
  u_pr, i_pr = _sc_gather(qu, qi, upair, ipair)
  return _mlp(u_pr, i_pr, hu, hi,
              W1[:EMBED], W1[EMBED:], b1.reshape(1, -1),
              W2, b2.reshape(1, -1), W3, b3.reshape(1, 1))


# speedup vs baseline: 2.2946x; 1.4434x over previous
import functools

import jax
import jax.numpy as jnp
from jax import lax
from jax.experimental import pallas as pl
from jax.experimental.pallas import tpu as pltpu
from jax.experimental.pallas import tpu_sc as plsc

BATCH = 16384
EMBED = 64
NC = 2
NS = 16
NW = NC * NS
B_PER_W = BATCH // NW
CHUNK = 128
N_CHUNKS = B_PER_W // CHUNK
PAIR = 4096


def _pack_body(ta_ref, tb_ref, out_ref):
  out_ref[...] = jnp.concatenate([ta_ref[...], tb_ref[...]], axis=0).T


def _pack(tab_t):
  n = tab_t.shape[1]
  nb = (n + 2 * PAIR - 1) // (2 * PAIR)
  last = (n + PAIR - 1) // PAIR - 1
  return pl.pallas_call(
      _pack_body,
      grid=(nb,),
      in_specs=[
          pl.BlockSpec((EMBED, PAIR), lambda m: (0, 2 * m)),
          pl.BlockSpec((EMBED, PAIR),
                       lambda m: (0, jnp.minimum(2 * m + 1, last))),
      ],
      out_specs=pl.BlockSpec((PAIR, 128), lambda m: (m, 0)),
      out_shape=jax.ShapeDtypeStruct((nb * PAIR, 128), jnp.float32),
  )(tab_t, tab_t)


def _sc_gather(qu2, qi2, upair, ipair):
  mesh = plsc.VectorSubcoreMesh(core_axis_name="c", subcore_axis_name="s")

  @functools.partial(
      pl.kernel,
      out_type=(
          jax.ShapeDtypeStruct((BATCH, 128), jnp.float32),
          jax.ShapeDtypeStruct((BATCH, 128), jnp.float32),
      ),
      mesh=mesh,
      scratch_types=[
          pltpu.VMEM((B_PER_W,), jnp.int32),
          pltpu.VMEM((B_PER_W, 128), jnp.float32),
          pltpu.SemaphoreType.DMA,
      ],
  )
  def k(qu_hbm, qi_hbm, up_hbm, ip_hbm, out_u, out_i, idx, rows, sem):
    wid = lax.axis_index("s") * NC + lax.axis_index("c")
    base = wid * B_PER_W
    for ids_hbm, pair_hbm, out in ((qu_hbm, up_hbm, out_u),
                                   (qi_hbm, ip_hbm, out_i)):
      pltpu.sync_copy(ids_hbm.at[pl.ds(base, B_PER_W)], idx)
      copies = []
      for j in range(B_PER_W // 16):
        iv = idx[pl.ds(j * 16, 16)]
        copies.append(pltpu.async_copy(
            pair_hbm.at[iv], rows.at[pl.ds(j * 16, 16)], sem))
      for c in copies:
        c.wait()
      pltpu.sync_copy(rows, out.at[pl.ds(base, B_PER_W)])

  return k(qu2, qi2, upair, ipair)


def _mlp_body(u, i, hu, hi, w1u, w1i, b1, w2, b2, w3, b3, out):
  f32 = jnp.float32
  hp = jax.lax.Precision.HIGHEST
  xu = jnp.where(hu[...] > 0.5, u[...][:, EMBED:], u[...][:, :EMBED])
  xi = jnp.where(hi[...] > 0.5, i[...][:, EMBED:], i[...][:, :EMBED])
  h = (jnp.dot(xu, w1u[...], preferred_element_type=f32, precision=hp)
       + jnp.dot(xi, w1i[...], preferred_element_type=f32, precision=hp)
       + b1[...])
  h = jnp.maximum(h, 0.0)
  h2 = jnp.dot(h, w2[...], preferred_element_type=f32, precision=hp) + b2[...]
  h2 = jnp.maximum(h2, 0.0)
  out[...] = jnp.dot(h2, w3[...], preferred_element_type=f32,
                     precision=hp) + b3[...]


def _mlp(u_pr, i_pr, hu, hi, W1u, W1i, b1, W2, b2, W3, b3, bm=2048):
  grid = (BATCH // bm,)
  full = lambda shape: pl.BlockSpec(shape, lambda m: (0,) * len(shape))
  return pl.pallas_call(
      _mlp_body,
      grid=grid,
      in_specs=[
          pl.BlockSpec((bm, 128), lambda m: (m, 0)),
          pl.BlockSpec((bm, 128), lambda m: (m, 0)),
          pl.BlockSpec((bm, 1), lambda m: (m, 0)),
          pl.BlockSpec((bm, 1), lambda m: (m, 0)),
          full((EMBED, 128)),
          full((EMBED, 128)),
          full((1, 128)),
          full((128, 64)),
          full((1, 64)),
          full((EMBED, 1)),
          full((1, 1)),
      ],
      out_specs=pl.BlockSpec((bm, 1), lambda m: (m, 0)),
      out_shape=jax.ShapeDtypeStruct((BATCH, 1), jnp.float32),
  )(u_pr, i_pr, hu, hi, W1u, W1i, b1, W2, b2, W3, b3)


def kernel(user_ids, item_ids, user_table, item_table, W1, b1, W2, b2, W3, b3):
  uid = user_ids.astype(jnp.int32)
  iid = item_ids.astype(jnp.int32)
  qu = (uid // (2 * PAIR)) * PAIR + uid % PAIR
  qi = (iid // (2 * PAIR)) * PAIR + iid % PAIR
  hu = ((uid // PAIR) % 2).astype(jnp.float32).reshape(-1, 1)
  hi = ((iid // PAIR) % 2).astype(jnp.float32).reshape(-1, 1)
  upair = _pack(user_table.T)
  ipair = _pack(item_table.T)
  u_pr, i_pr = _sc_gather(qu, qi, upair, ipair)
  return _mlp(u_pr, i_pr, hu, hi,
              W1[:EMBED], W1[EMBED:], b1.reshape(1, -1),
              W2, b2.reshape(1, -1), W3, b3.reshape(1, 1))
